# baseline (speedup 1.0000x reference)
import functools

import jax
import jax.numpy as jnp
import numpy as np
from jax import lax
from jax.experimental import pallas as pl
from jax.experimental.pallas import tpu as pltpu
from jax.experimental.pallas import tpu_sc as plsc

_N_RET, _N_LGN, _N_V1, _N_IT = 128, 128, 256, 128
_C, _H, _W = 3, 224, 224
_B = 256
_N_CLASSES = 1000
_K = _N_RET * 16
_CHW = _C * _H * _W
_INV_T = 1.25

_NC, _NS = 2, 16
_NW = _NC * _NS
_RPW = _K // _NW


def _sc_gather(x_cols, pm_flat):
    mesh = plsc.VectorSubcoreMesh(core_axis_name="c", subcore_axis_name="s")

    @functools.partial(
        pl.kernel,
        out_type=jax.ShapeDtypeStruct((_K, _B), jnp.float32),
        mesh=mesh,
        scratch_types=[
            pltpu.VMEM((_RPW,), jnp.int32),
            pltpu.VMEM((_RPW, _B), jnp.float32),
            pltpu.SemaphoreType.DMA,
        ],
    )
    def gather_kernel(x_hbm, pm_hbm, out_hbm, idx_v, rows_v, sem):
        wid = lax.axis_index("s") * _NC + lax.axis_index("c")
        sl = pl.ds(wid * _RPW, _RPW)
        pltpu.sync_copy(pm_hbm.at[sl], idx_v)
        pltpu.async_copy(x_hbm.at[idx_v], rows_v, sem).wait()
        pltpu.sync_copy(rows_v, out_hbm.at[sl])

    return gather_kernel(x_cols, pm_flat)


def _dot(a, b, dims):
    return lax.dot_general(a, b, (dims, ((), ())),
                           preferred_element_type=jnp.float32)


def _soma(inp, w, b):
    sw = jnp.sum(w, axis=-1)
    dend = jnp.tanh(inp[:, None, :] * sw[:, :, None] + b[:, :, None])
    return jax.nn.sigmoid(jnp.sum(dend, axis=1))


def _pool(r, mb):
    m = mb.astype(jnp.float32)
    eff = jnp.where(jnp.sum(m, axis=1, keepdims=True) > 0.5, m, 1.0)
    e = jnp.exp(r * _INV_T)
    return (_dot(eff, e * r, ((1,), (0,)))
            / _dot(eff, e, ((1,), (0,))))


def _tc_body(p_ref, wr_ref, br_ref, wl_ref, bl_ref, wv_ref, bv_ref,
             wi_ref, bi_ref, wc_ref, bc_ref, m1_ref, m2_ref, out_ref):
    p4 = p_ref[...].reshape(_N_RET, 4, 4, _B)
    t = p4 * wr_ref[...][:, :, :, None]
    dend = jnp.tanh(jnp.sum(t, axis=2) + br_ref[...][:, :, None])
    r1 = jax.nn.sigmoid(jnp.sum(dend, axis=1))

    r2 = _soma(r1, wl_ref[...], bl_ref[...])
    v1 = _pool(r2, m1_ref[...])
    r3 = _soma(v1, wv_ref[...], bv_ref[...])
    it = _pool(r3, m2_ref[...])
    r4 = _soma(it, wi_ref[...], bi_ref[...])

    out_ref[...] = _dot(r4, wc_ref[...], ((0,), (1,))) + bc_ref[...]


def kernel(x, w_retina, b_retina, w_lgn, b_lgn, w_v1, b_v1, w_it, b_it,
           Wc, bc, pixel_map, lgn_to_v1, v1_to_it):
    x_cols = x.transpose(1, 2, 3, 0).reshape(_CHW, _B)
    p = _sc_gather(x_cols, pixel_map.reshape(-1))

    return pl.pallas_call(
        _tc_body,
        out_shape=jax.ShapeDtypeStruct((_B, _N_CLASSES), jnp.float32),
    )(p, w_retina, b_retina, w_lgn, b_lgn, w_v1, b_v1, w_it, b_it,
      Wc, bc.reshape(1, _N_CLASSES), lgn_to_v1, v1_to_it)


# speedup vs baseline: 1.0020x; 1.0020x over previous
import functools

import jax
import jax.numpy as jnp
from jax import lax
from jax.experimental import pallas as pl
from jax.experimental.pallas import tpu as pltpu
from jax.experimental.pallas import tpu_sc as plsc

_N_RET, _N_LGN, _N_V1, _N_IT = 128, 128, 256, 128
_C, _H, _W = 3, 224, 224
_B = 256
_N_CLASSES = 1000
_K = _N_RET * 16
_CHW = _C * _H * _W
_INV_T = 1.25

_NC, _NS = 2, 16
_NW = _NC * _NS
_RPW = _K // _NW


def _sc_gather(x_cols, pm_flat):
    mesh = plsc.VectorSubcoreMesh(core_axis_name="c", subcore_axis_name="s")

    @functools.partial(
        pl.kernel,
        out_type=jax.ShapeDtypeStruct((_K, _B), jnp.float32),
        mesh=mesh,
        scratch_types=[
            pltpu.VMEM((_RPW,), jnp.int32),
            pltpu.VMEM((_RPW, _B), jnp.float32),
            pltpu.SemaphoreType.DMA,
        ],
    )
    def gather_kernel(x_hbm, pm_hbm, out_hbm, idx_v, rows_v, sem):
        wid = lax.axis_index("s") * _NC + lax.axis_index("c")
        sl = pl.ds(wid * _RPW, _RPW)
        pltpu.sync_copy(pm_hbm.at[sl], idx_v)
        pltpu.async_copy(x_hbm.at[idx_v], rows_v, sem).wait()
        pltpu.sync_copy(rows_v, out_hbm.at[sl])

    return gather_kernel(x_cols, pm_flat)


def _dot(a, b, dims):
    return lax.dot_general(a, b, (dims, ((), ())),
                           preferred_element_type=jnp.float32)


def _soma(inp, w, b):
    sw = jnp.sum(w, axis=-1)
    dend = jnp.tanh(inp[:, None, :] * sw[:, :, None] + b[:, :, None])
    return jax.nn.sigmoid(jnp.sum(dend, axis=1))


def _pool(r, mb):
    m = mb.astype(jnp.float32)
    eff = jnp.where(jnp.sum(m, axis=1, keepdims=True) > 0.5, m, 1.0)
    e = jnp.exp(r * _INV_T)
    return (_dot(eff, e * r, ((1,), (0,)))
            / _dot(eff, e, ((1,), (0,))))


def _tc_body(p_ref, wr_ref, br_ref, wl_ref, bl_ref, wv_ref, bv_ref,
             wi_ref, bi_ref, wc_ref, bc_ref, m1_ref, m2_ref, out_ref):
    p4 = p_ref[...].reshape(_N_RET, 4, 4, _B)
    t = p4 * wr_ref[...][:, :, :, None]
    dend = jnp.tanh(jnp.sum(t, axis=2) + br_ref[...][:, :, None])
    r1 = jax.nn.sigmoid(jnp.sum(dend, axis=1))

    r2 = _soma(r1, wl_ref[...], bl_ref[...])
    v1 = _pool(r2, m1_ref[...])
    r3 = _soma(v1, wv_ref[...], bv_ref[...])
    it = _pool(r3, m2_ref[...])
    r4 = _soma(it, wi_ref[...], bi_ref[...])

    out_ref[...] = _dot(r4, wc_ref[...], ((0,), (1,))) + bc_ref[...]


def kernel(x, w_retina, b_retina, w_lgn, b_lgn, w_v1, b_v1, w_it, b_it,
           Wc, bc, pixel_map, lgn_to_v1, v1_to_it):
    x_cols = x.transpose(1, 2, 3, 0).reshape(_CHW, _B)
    p = _sc_gather(x_cols, pixel_map.reshape(-1))

    return pl.pallas_call(
        _tc_body,
        out_shape=jax.ShapeDtypeStruct((_B, _N_CLASSES), jnp.float32),
    )(p, w_retina, b_retina, w_lgn, b_lgn, w_v1, b_v1, w_it, b_it,
      Wc, bc.reshape(1, _N_CLASSES), lgn_to_v1, v1_to_it)


# speedup vs baseline: 1.0422x; 1.0401x over previous
import functools

import jax
import jax.numpy as jnp
from jax import lax
from jax.experimental import pallas as pl
from jax.experimental.pallas import tpu as pltpu
from jax.experimental.pallas import tpu_sc as plsc

_N_RET, _N_LGN, _N_V1, _N_IT = 128, 128, 256, 128
_C, _H, _W = 3, 224, 224
_B = 256
_N_CLASSES = 1000
_K = _N_RET * 16
_CHW = _C * _H * _W
_INV_T = 1.25

_NC, _NS = 1, 16
_NW = _NC * _NS
_RPW = _K // _NW


def _sc_gather(x_cols, pm_flat):
    mesh = plsc.VectorSubcoreMesh(core_axis_name="c", subcore_axis_name="s",
                                  num_cores=1)

    @functools.partial(
        pl.kernel,
        out_type=jax.ShapeDtypeStruct((_K, _B), jnp.float32),
        mesh=mesh,
        scratch_types=[
            pltpu.VMEM((_RPW,), jnp.int32),
            pltpu.VMEM((_RPW, _B), jnp.float32),
            pltpu.SemaphoreType.DMA,
        ],
    )
    def gather_kernel(x_hbm, pm_hbm, out_hbm, idx_v, rows_v, sem):
        wid = lax.axis_index("s") * _NC + lax.axis_index("c")
        sl = pl.ds(wid * _RPW, _RPW)
        pltpu.sync_copy(pm_hbm.at[sl], idx_v)
        pltpu.async_copy(x_hbm.at[idx_v], rows_v, sem).wait()
        pltpu.sync_copy(rows_v, out_hbm.at[sl])

    return gather_kernel(x_cols, pm_flat)


def _dot(a, b, dims):
    return lax.dot_general(a, b, (dims, ((), ())),
                           preferred_element_type=jnp.float32)


def _soma(inp, w, b):
    sw = jnp.sum(w, axis=-1)
    dend = jnp.tanh(inp[:, None, :] * sw[:, :, None] + b[:, :, None])
    return jax.nn.sigmoid(jnp.sum(dend, axis=1))


def _pool(r, mb):
    m = mb.astype(jnp.float32)
    eff = jnp.where(jnp.sum(m, axis=1, keepdims=True) > 0.5, m, 1.0)
    e = jnp.exp(r * _INV_T)
    return (_dot(eff, e * r, ((1,), (0,)))
            / _dot(eff, e, ((1,), (0,))))


def _tc_body(p_ref, wr_ref, br_ref, wl_ref, bl_ref, wv_ref, bv_ref,
             wi_ref, bi_ref, wc_ref, bc_ref, m1_ref, m2_ref, out_ref):
    p4 = p_ref[...].reshape(_N_RET, 4, 4, _B)
    t = p4 * wr_ref[...][:, :, :, None]
    dend = jnp.tanh(jnp.sum(t, axis=2) + br_ref[...][:, :, None])
    r1 = jax.nn.sigmoid(jnp.sum(dend, axis=1))

    r2 = _soma(r1, wl_ref[...], bl_ref[...])
    v1 = _pool(r2, m1_ref[...])
    r3 = _soma(v1, wv_ref[...], bv_ref[...])
    it = _pool(r3, m2_ref[...])
    r4 = _soma(it, wi_ref[...], bi_ref[...])

    out_ref[...] = _dot(r4, wc_ref[...], ((0,), (1,))) + bc_ref[...]


def kernel(x, w_retina, b_retina, w_lgn, b_lgn, w_v1, b_v1, w_it, b_it,
           Wc, bc, pixel_map, lgn_to_v1, v1_to_it):
    x_cols = x.transpose(1, 2, 3, 0).reshape(_CHW, _B)
    p = _sc_gather(x_cols, pixel_map.reshape(-1))

    return pl.pallas_call(
        _tc_body,
        out_shape=jax.ShapeDtypeStruct((_B, _N_CLASSES), jnp.float32),
    )(p, w_retina, b_retina, w_lgn, b_lgn, w_v1, b_v1, w_it, b_it,
      Wc, bc.reshape(1, _N_CLASSES), lgn_to_v1, v1_to_it)
